# baseline (speedup 1.0000x reference)
import jax
import jax.numpy as jnp
from jax.experimental import pallas as pl
from jax.experimental.pallas import tpu as pltpu

_STRIDE = 2
_OUT_G = 32
_NUM_SEG = _OUT_G ** 3
_C = 128

_P = 10000
_U = 8
_CORES = 2


def _scatter_kernel(seg_hbm, feats_ref, out_ref, seg_smem, sems):
    c = pl.program_id(0)
    j = pl.program_id(1)
    tiles_per_core = pl.num_programs(1)
    slot = jax.lax.rem(j, 2)

    @pl.when(j == 0)
    def _():
        out_ref[...] = jnp.full(out_ref.shape, -jnp.inf, jnp.float32)
        t0 = c * tiles_per_core
        pltpu.make_async_copy(seg_hbm.at[t0], seg_smem.at[0], sems.at[0]).start()

    @pl.when(j + 1 < tiles_per_core)
    def _():
        t1 = c * tiles_per_core + j + 1
        nslot = jax.lax.rem(j + 1, 2)
        pltpu.make_async_copy(
            seg_hbm.at[t1], seg_smem.at[nslot], sems.at[nslot]
        ).start()

    t = c * tiles_per_core + j
    pltpu.make_async_copy(seg_hbm.at[t], seg_smem.at[slot], sems.at[slot]).wait()

    def body(it, carry):
        base = it * _U
        for u in range(_U):
            k = base + u
            s = seg_smem[slot, k]
            out_ref[s, 0] = jnp.maximum(out_ref[s, 0], feats_ref[k, 0])
        return carry

    jax.lax.fori_loop(0, _P // _U, body, 0)


def _combine_kernel(p_ref, o_ref):
    m = jnp.maximum(p_ref[0], p_ref[1])
    o_ref[...] = jnp.where(m == -jnp.inf, jnp.float32(0.0), m)


def kernel(feats, coords):
    n = feats.shape[0]
    cell = coords // _STRIDE
    seg = (cell[:, 0] * _OUT_G + cell[:, 1]) * _OUT_G + cell[:, 2]
    seg = seg.astype(jnp.int32)

    n_tiles = n // _P
    tiles_per_core = n_tiles // _CORES
    seg2d = seg.reshape(n_tiles, _P)
    feats3d = feats.reshape(n, 1, _C)

    partials = pl.pallas_call(
        _scatter_kernel,
        out_shape=jax.ShapeDtypeStruct((_CORES, _NUM_SEG, 1, _C), jnp.float32),
        grid=(_CORES, tiles_per_core),
        in_specs=[
            pl.BlockSpec(memory_space=pl.ANY),
            pl.BlockSpec((_P, 1, _C), lambda c, j: (c * (n // _P // _CORES) + j, 0, 0)),
        ],
        out_specs=pl.BlockSpec((None, _NUM_SEG, 1, _C), lambda c, j: (c, 0, 0, 0)),
        scratch_shapes=[
            pltpu.SMEM((2, _P), jnp.int32),
            pltpu.SemaphoreType.DMA((2,)),
        ],
        compiler_params=pltpu.CompilerParams(
            dimension_semantics=("parallel", "arbitrary"),
            vmem_limit_bytes=56 * 1024 * 1024,
        ),
        name="sparse_pool_scatter",
    )(seg2d, feats3d)

    parts = partials.reshape(_CORES, _NUM_SEG, _C)
    n_blk = 4
    sb = _NUM_SEG // (_CORES * n_blk)
    pooled = pl.pallas_call(
        _combine_kernel,
        out_shape=jax.ShapeDtypeStruct((_NUM_SEG, _C), jnp.float32),
        grid=(_CORES, n_blk),
        in_specs=[
            pl.BlockSpec((_CORES, sb, _C), lambda c, j: (0, c * n_blk + j, 0)),
        ],
        out_specs=pl.BlockSpec((sb, _C), lambda c, j: (c * n_blk + j, 0)),
        compiler_params=pltpu.CompilerParams(
            dimension_semantics=("parallel", "arbitrary"),
        ),
        name="sparse_pool_combine",
    )(parts)
    return pooled
